# baseline (speedup 1.0000x reference)
import functools

import jax
import jax.numpy as jnp
from jax import lax
from jax.experimental import pallas as pl
from jax.experimental.pallas import tpu as pltpu
from jax.experimental.pallas import tpu_sc as plsc

N = 10000
D = 256
H = 256
HH = H // 2

NSUB = 16
SP_ROWS = 10240
K = 64
NBUF = 4
NIDX = 8
NCHUNK = 160
NCHUNK_P = NCHUNK + NIDX
EPT = K * NCHUNK


def _dotT(a, b):
    return lax.dot_general(a, b, (((1,), (1,)), ((), ())),
                           preferred_element_type=jnp.float32)


def _bn_scale_shift(g, b, rm, rv):
    s = g / jnp.sqrt(rv + 1e-5)
    return s, b - rm * s


def _stage_a_body(x, fc0w, fc0b, g0, b0, rm0, rv0,
                  gin0w, gin0b, gin1w, gin1b, ww, wb,
                  hl_out, hr_out, wc0_out, bc0_out, wc1_out, bc1_out):
    s0, t0 = _bn_scale_shift(g0[...], b0[...], rm0[...], rv0[...])
    z = _dotT(x[...], fc0w[...]) + fc0b[...]
    h = jnp.maximum(z * s0 + t0, 0.0)
    hl_out[...] = h[:, :HH]
    hr_out[...] = h[:, HH:]
    wc0_out[...] = jnp.dot(ww[...], gin0w[...],
                           preferred_element_type=jnp.float32)
    bc0_out[...] = _dotT(gin0b[...], ww[...]) + wb[...]
    wc1_out[...] = jnp.dot(ww[...], gin1w[...],
                           preferred_element_type=jnp.float32)
    bc1_out[...] = _dotT(gin1b[...], ww[...]) + wb[...]


def _stage_a(x, fc0w, fc0b, g0, b0, rm0, rv0,
             gin0w, gin0b, gin1w, gin1b, ww, wb):
    f32 = jnp.float32
    return pl.pallas_call(
        _stage_a_body,
        out_shape=(
            jax.ShapeDtypeStruct((N, HH), f32),
            jax.ShapeDtypeStruct((N, HH), f32),
            jax.ShapeDtypeStruct((H, H), f32),
            jax.ShapeDtypeStruct((1, H), f32),
            jax.ShapeDtypeStruct((H, H), f32),
            jax.ShapeDtypeStruct((1, H), f32),
        ),
    )(x, fc0w, fc0b, g0, b0, rm0, rv0, gin0w, gin0b, gin1w, gin1b, ww, wb)


def _gin_stage_body(hl, hr, aggl, aggr, wc, bc, eps, g, b, rm, rv, *outs):
    e = eps[0, 0]
    yl = (1.0 + e) * hl[...] + aggl[...]
    yr = (1.0 + e) * hr[...] + aggr[...]
    w = wc[...]
    z = _dotT(yl, w[:, :HH]) + _dotT(yr, w[:, HH:]) + bc[...]
    s, t = _bn_scale_shift(g[...], b[...], rm[...], rv[...])
    h = jnp.maximum(z * s + t, 0.0)
    if len(outs) == 2:
        outs[0][...] = h[:, :HH]
        outs[1][...] = h[:, HH:]
    else:
        outs[0][...] = h


def _gin_stage(hl, hr, aggl, aggr, wc, bc, eps, g, b, rm, rv, last):
    f32 = jnp.float32
    if last:
        out_shape = jax.ShapeDtypeStruct((N, H), f32)
    else:
        out_shape = (jax.ShapeDtypeStruct((N, HH), f32),
                     jax.ShapeDtypeStruct((N, HH), f32))
    return pl.pallas_call(
        _gin_stage_body,
        out_shape=out_shape,
    )(hl, hr, aggl, aggr, wc, bc, eps, g, b, rm, rv)


def _segsum_body(hl_hbm, hr_hbm, src_hbm, dst_hbm, outl_hbm, outr_hbm,
                 idxs_v, idxd_v, rows_v, zrow_v, semi, semg, acc_sh):
    c = lax.axis_index("c")
    s = lax.axis_index("s")

    zv = jnp.zeros((16,), jnp.float32)
    for r in range(32):
        for j in range(HH // 16):
            zrow_v[r, pl.ds(j * 16, 16)] = zv
    base = s * 640
    for i in range(20):
        pltpu.sync_copy(zrow_v, acc_sh.at[pl.ds(base + i * 32, 32)])

    plsc.subcore_barrier()

    def idx_start(j, m):
        pltpu.async_copy(src_hbm.at[s, j], idxs_v.at[m], semi[m])
        pltpu.async_copy(dst_hbm.at[s, j], idxd_v.at[m], semi[m])

    def idx_wait(m):
        pltpu.make_async_copy(src_hbm.at[s, 0], idxs_v.at[m],
                              semi[m]).wait()
        pltpu.make_async_copy(dst_hbm.at[s, 0], idxd_v.at[m],
                              semi[m]).wait()

    def run_edges(h_hbm):
        for m in range(NIDX):
            idx_start(m, m)
        for q in range(NBUF):
            idx_wait(q)
            pltpu.async_copy(h_hbm.at[idxs_v.at[q]], rows_v.at[q], semg[q])

        def outer(jj, carry):
            j0 = NIDX * jj
            for u in range(NIDX):
                j = j0 + u
                q = u % NBUF
                m = u
                pltpu.make_async_copy(h_hbm.at[idxs_v.at[m]], rows_v.at[q],
                                      semg[q]).wait()
                pltpu.sync_copy(rows_v.at[q], acc_sh.at[idxd_v.at[m]],
                                add=True)
                idx_start(j + NIDX, m)
                nm = (u + NBUF) % NIDX
                idx_wait(nm)
                pltpu.async_copy(h_hbm.at[idxs_v.at[nm]], rows_v.at[q],
                                 semg[q])
            return carry

        lax.fori_loop(0, NCHUNK // NIDX, outer, 0)
        for q in range(NBUF):
            pltpu.make_async_copy(h_hbm.at[idxs_v.at[0]], rows_v.at[q],
                                  semg[q]).wait()
        for m in range(NBUF, NIDX):
            idx_wait(m)

    @pl.when(c == 0)
    def _():
        run_edges(hl_hbm)

    @pl.when(c == 1)
    def _():
        run_edges(hr_hbm)

    plsc.subcore_barrier()

    def copy_out(out_hbm):
        @pl.when(s < 15)
        def _():
            st = s * 640
            pltpu.sync_copy(acc_sh.at[pl.ds(st, 640)],
                            out_hbm.at[pl.ds(st, 640)])

        @pl.when(s == 15)
        def _():
            pltpu.sync_copy(acc_sh.at[pl.ds(9600, 400)],
                            out_hbm.at[pl.ds(9600, 400)])

    @pl.when(c == 0)
    def _():
        copy_out(outl_hbm)

    @pl.when(c == 1)
    def _():
        copy_out(outr_hbm)


@functools.partial(
    pl.kernel,
    out_type=(jax.ShapeDtypeStruct((N, HH), jnp.float32),
              jax.ShapeDtypeStruct((N, HH), jnp.float32)),
    mesh=plsc.VectorSubcoreMesh(core_axis_name="c", subcore_axis_name="s"),
    scratch_types=(
        [
            pltpu.VMEM((NIDX, K), jnp.int32),
            pltpu.VMEM((NIDX, K), jnp.int32),
            pltpu.VMEM((NBUF, K, HH), jnp.float32),
            pltpu.VMEM((32, HH), jnp.float32),
        ]
        + [pltpu.SemaphoreType.DMA] * (NIDX + NBUF)
        + [pltpu.VMEM_SHARED((SP_ROWS, HH), jnp.float32)]
    ),
)
def _segsum(hl, hr, src3, dst3, outl, outr,
            idxs_v, idxd_v, rows_v, zrow_v, *rest):
    semi = rest[:NIDX]
    semg = rest[NIDX:NIDX + NBUF]
    acc_sh = rest[-1]
    _segsum_body(hl, hr, src3, dst3, outl, outr,
                 idxs_v, idxd_v, rows_v, zrow_v, semi, semg, acc_sh)


def kernel(x, edge_index, fc0_W, fc0_b, gin0_W, gin0_b, gin1_W, gin1_b,
           W_W, W_b, eps0, eps1, bn0_g, bn0_b, bn0_rm, bn0_rv,
           bn1_g, bn1_b, bn1_rm, bn1_rv, bn2_g, bn2_b, bn2_rm, bn2_rv):
    f32 = jnp.float32
    row = lambda v: v.reshape(1, H).astype(f32)

    E = edge_index.shape[1]
    pad = NSUB * EPT - E
    src = jnp.concatenate([edge_index[0], jnp.zeros((pad,), jnp.int32)])
    dst = jnp.concatenate([edge_index[1], jnp.full((pad,), N, jnp.int32)])
    src3 = jnp.concatenate(
        [src.reshape(NSUB, NCHUNK, K),
         jnp.zeros((NSUB, NCHUNK_P - NCHUNK, K), jnp.int32)], axis=1)
    dst3 = jnp.concatenate(
        [dst.reshape(NSUB, NCHUNK, K),
         jnp.full((NSUB, NCHUNK_P - NCHUNK, K), N, jnp.int32)], axis=1)

    h0l, h0r, wc0, bc0, wc1, bc1 = _stage_a(
        x, fc0_W, row(fc0_b), row(bn0_g), row(bn0_b), row(bn0_rm),
        row(bn0_rv), gin0_W, row(gin0_b), gin1_W, row(gin1_b),
        W_W, row(W_b))

    agg0l, agg0r = _segsum(h0l, h0r, src3, dst3)
    h1l, h1r = _gin_stage(h0l, h0r, agg0l, agg0r, wc0, bc0,
                          eps0.reshape(1, 1), row(bn1_g), row(bn1_b),
                          row(bn1_rm), row(bn1_rv), last=False)

    agg1l, agg1r = _segsum(h1l, h1r, src3, dst3)
    h2 = _gin_stage(h1l, h1r, agg1l, agg1r, wc1, bc1,
                    eps1.reshape(1, 1), row(bn2_g), row(bn2_b),
                    row(bn2_rm), row(bn2_rv), last=True)
    return h2


# speedup vs baseline: 1.1126x; 1.1126x over previous
import functools

import jax
import jax.numpy as jnp
from jax import lax
from jax.experimental import pallas as pl
from jax.experimental.pallas import tpu as pltpu
from jax.experimental.pallas import tpu_sc as plsc

N = 10000
D = 256
H = 256
HH = H // 2

NSUB = 16
SP_ROWS = 10240
K = 128
NCHUNK = 80
NCHUNK_P = 84
EPT = K * NCHUNK


def _dotT(a, b):
    return lax.dot_general(a, b, (((1,), (1,)), ((), ())),
                           preferred_element_type=jnp.float32)


def _bn_scale_shift(g, b, rm, rv):
    s = g / jnp.sqrt(rv + 1e-5)
    return s, b - rm * s


def _stage_a_body(x, fc0w, fc0b, g0, b0, rm0, rv0,
                  gin0w, gin0b, gin1w, gin1b, ww, wb,
                  hl_out, hr_out, wc0_out, bc0_out, wc1_out, bc1_out):
    s0, t0 = _bn_scale_shift(g0[...], b0[...], rm0[...], rv0[...])
    z = _dotT(x[...], fc0w[...]) + fc0b[...]
    h = jnp.maximum(z * s0 + t0, 0.0)
    hl_out[...] = h[:, :HH]
    hr_out[...] = h[:, HH:]
    wc0_out[...] = jnp.dot(ww[...], gin0w[...],
                           preferred_element_type=jnp.float32)
    bc0_out[...] = _dotT(gin0b[...], ww[...]) + wb[...]
    wc1_out[...] = jnp.dot(ww[...], gin1w[...],
                           preferred_element_type=jnp.float32)
    bc1_out[...] = _dotT(gin1b[...], ww[...]) + wb[...]


def _stage_a(x, fc0w, fc0b, g0, b0, rm0, rv0,
             gin0w, gin0b, gin1w, gin1b, ww, wb):
    f32 = jnp.float32
    return pl.pallas_call(
        _stage_a_body,
        out_shape=(
            jax.ShapeDtypeStruct((N, HH), f32),
            jax.ShapeDtypeStruct((N, HH), f32),
            jax.ShapeDtypeStruct((H, H), f32),
            jax.ShapeDtypeStruct((1, H), f32),
            jax.ShapeDtypeStruct((H, H), f32),
            jax.ShapeDtypeStruct((1, H), f32),
        ),
    )(x, fc0w, fc0b, g0, b0, rm0, rv0, gin0w, gin0b, gin1w, gin1b, ww, wb)


def _gin_stage_body(hl, hr, aggl, aggr, wc, bc, eps, g, b, rm, rv, *outs):
    e = eps[0, 0]
    yl = (1.0 + e) * hl[...] + aggl[...]
    yr = (1.0 + e) * hr[...] + aggr[...]
    w = wc[...]
    z = _dotT(yl, w[:, :HH]) + _dotT(yr, w[:, HH:]) + bc[...]
    s, t = _bn_scale_shift(g[...], b[...], rm[...], rv[...])
    h = jnp.maximum(z * s + t, 0.0)
    if len(outs) == 2:
        outs[0][...] = h[:, :HH]
        outs[1][...] = h[:, HH:]
    else:
        outs[0][...] = h


def _gin_stage(hl, hr, aggl, aggr, wc, bc, eps, g, b, rm, rv, last):
    f32 = jnp.float32
    if last:
        out_shape = jax.ShapeDtypeStruct((N, H), f32)
    else:
        out_shape = (jax.ShapeDtypeStruct((N, HH), f32),
                     jax.ShapeDtypeStruct((N, HH), f32))
    return pl.pallas_call(
        _gin_stage_body,
        out_shape=out_shape,
    )(hl, hr, aggl, aggr, wc, bc, eps, g, b, rm, rv)


def _segsum_body(hl_hbm, hr_hbm, src_hbm, dst_hbm, outl_hbm, outr_hbm,
                 idxs_v, idxd_v, rows_v, zrow_v, semi, semg, acc_sh):
    c = lax.axis_index("c")
    s = lax.axis_index("s")

    def idx_start(j, m):
        pltpu.async_copy(src_hbm.at[s, j], idxs_v.at[m], semi[m])
        pltpu.async_copy(dst_hbm.at[s, j], idxd_v.at[m], semi[m])

    def idx_wait(m):
        pltpu.make_async_copy(src_hbm.at[s, 0], idxs_v.at[m],
                              semi[m]).wait()
        pltpu.make_async_copy(dst_hbm.at[s, 0], idxd_v.at[m],
                              semi[m]).wait()

    def prologue(h_hbm):
        for m in range(4):
            idx_start(m, m)
        for b in range(2):
            idx_wait(b)
            pltpu.async_copy(h_hbm.at[idxs_v.at[b]], rows_v.at[b], semg[b])

    @pl.when(c == 0)
    def _():
        prologue(hl_hbm)

    @pl.when(c == 1)
    def _():
        prologue(hr_hbm)

    zv = jnp.zeros((16,), jnp.float32)
    for r in range(32):
        for j in range(HH // 16):
            zrow_v[r, pl.ds(j * 16, 16)] = zv
    base = s * 640
    for i in range(20):
        pltpu.sync_copy(zrow_v, acc_sh.at[pl.ds(base + i * 32, 32)])

    plsc.subcore_barrier()

    def run_edges(h_hbm):
        def outer(jj, carry):
            j0 = 4 * jj
            for u in range(4):
                j = j0 + u
                b = u % 2
                m = u
                pltpu.make_async_copy(h_hbm.at[idxs_v.at[m]], rows_v.at[b],
                                      semg[b]).wait()
                pltpu.sync_copy(rows_v.at[b], acc_sh.at[idxd_v.at[m]],
                                add=True)
                idx_start(j + 4, m)
                nm = (u + 2) % 4
                idx_wait(nm)
                pltpu.async_copy(h_hbm.at[idxs_v.at[nm]], rows_v.at[b],
                                 semg[b])
            return carry

        lax.fori_loop(0, NCHUNK // 4, outer, 0)
        for b in range(2):
            pltpu.make_async_copy(h_hbm.at[idxs_v.at[0]], rows_v.at[b],
                                  semg[b]).wait()
        for m in (2, 3):
            idx_wait(m)

    @pl.when(c == 0)
    def _():
        run_edges(hl_hbm)

    @pl.when(c == 1)
    def _():
        run_edges(hr_hbm)

    plsc.subcore_barrier()

    def copy_out(out_hbm):
        @pl.when(s < 15)
        def _():
            st = s * 640
            pltpu.sync_copy(acc_sh.at[pl.ds(st, 640)],
                            out_hbm.at[pl.ds(st, 640)])

        @pl.when(s == 15)
        def _():
            pltpu.sync_copy(acc_sh.at[pl.ds(9600, 400)],
                            out_hbm.at[pl.ds(9600, 400)])

    @pl.when(c == 0)
    def _():
        copy_out(outl_hbm)

    @pl.when(c == 1)
    def _():
        copy_out(outr_hbm)


@functools.partial(
    pl.kernel,
    out_type=(jax.ShapeDtypeStruct((N, HH), jnp.float32),
              jax.ShapeDtypeStruct((N, HH), jnp.float32)),
    mesh=plsc.VectorSubcoreMesh(core_axis_name="c", subcore_axis_name="s"),
    scratch_types=(
        [
            pltpu.VMEM((4, K), jnp.int32),
            pltpu.VMEM((4, K), jnp.int32),
            pltpu.VMEM((2, K, HH), jnp.float32),
            pltpu.VMEM((32, HH), jnp.float32),
        ]
        + [pltpu.SemaphoreType.DMA] * 6
        + [pltpu.VMEM_SHARED((SP_ROWS, HH), jnp.float32)]
    ),
)
def _segsum(hl, hr, src3, dst3, outl, outr,
            idxs_v, idxd_v, rows_v, zrow_v, *rest):
    semi = rest[:4]
    semg = rest[4:6]
    acc_sh = rest[-1]
    _segsum_body(hl, hr, src3, dst3, outl, outr,
                 idxs_v, idxd_v, rows_v, zrow_v, semi, semg, acc_sh)


def kernel(x, edge_index, fc0_W, fc0_b, gin0_W, gin0_b, gin1_W, gin1_b,
           W_W, W_b, eps0, eps1, bn0_g, bn0_b, bn0_rm, bn0_rv,
           bn1_g, bn1_b, bn1_rm, bn1_rv, bn2_g, bn2_b, bn2_rm, bn2_rv):
    f32 = jnp.float32
    row = lambda v: v.reshape(1, H).astype(f32)

    E = edge_index.shape[1]
    pad = NSUB * EPT - E
    src = jnp.concatenate([edge_index[0], jnp.zeros((pad,), jnp.int32)])
    dst = jnp.concatenate([edge_index[1], jnp.full((pad,), N, jnp.int32)])
    src3 = jnp.concatenate(
        [src.reshape(NSUB, NCHUNK, K),
         jnp.zeros((NSUB, NCHUNK_P - NCHUNK, K), jnp.int32)], axis=1)
    dst3 = jnp.concatenate(
        [dst.reshape(NSUB, NCHUNK, K),
         jnp.full((NSUB, NCHUNK_P - NCHUNK, K), N, jnp.int32)], axis=1)

    h0l, h0r, wc0, bc0, wc1, bc1 = _stage_a(
        x, fc0_W, row(fc0_b), row(bn0_g), row(bn0_b), row(bn0_rm),
        row(bn0_rv), gin0_W, row(gin0_b), gin1_W, row(gin1_b),
        W_W, row(W_b))

    agg0l, agg0r = _segsum(h0l, h0r, src3, dst3)
    h1l, h1r = _gin_stage(h0l, h0r, agg0l, agg0r, wc0, bc0,
                          eps0.reshape(1, 1), row(bn1_g), row(bn1_b),
                          row(bn1_rm), row(bn1_rv), last=False)

    agg1l, agg1r = _segsum(h1l, h1r, src3, dst3)
    h2 = _gin_stage(h1l, h1r, agg1l, agg1r, wc1, bc1,
                    eps1.reshape(1, 1), row(bn2_g), row(bn2_b),
                    row(bn2_rm), row(bn2_rv), last=True)
    return h2


# speedup vs baseline: 1.3391x; 1.2036x over previous
import functools

import jax
import jax.numpy as jnp
from jax import lax
from jax.experimental import pallas as pl
from jax.experimental.pallas import tpu as pltpu
from jax.experimental.pallas import tpu_sc as plsc

N = 10000
D = 256
H = 256
HH = H // 2

NSUB = 16
SP_ROWS = 10240
K = 128
NCHUNK = 80
NCHUNK_P = 82
EPT = K * NCHUNK


def _dotT(a, b):
    return lax.dot_general(a, b, (((1,), (1,)), ((), ())),
                           preferred_element_type=jnp.float32)


def _bn_scale_shift(g, b, rm, rv):
    s = g / jnp.sqrt(rv + 1e-5)
    return s, b - rm * s


def _stage_a_body(x, fc0w, fc0b, g0, b0, rm0, rv0,
                  gin0w, gin0b, gin1w, gin1b, ww, wb,
                  hl_out, hr_out, wc0_out, bc0_out, wc1_out, bc1_out):
    s0, t0 = _bn_scale_shift(g0[...], b0[...], rm0[...], rv0[...])
    z = _dotT(x[...], fc0w[...]) + fc0b[...]
    h = jnp.maximum(z * s0 + t0, 0.0)
    hl_out[...] = h[:, :HH]
    hr_out[...] = h[:, HH:]
    wc0_out[...] = jnp.dot(ww[...], gin0w[...],
                           preferred_element_type=jnp.float32)
    bc0_out[...] = _dotT(gin0b[...], ww[...]) + wb[...]
    wc1_out[...] = jnp.dot(ww[...], gin1w[...],
                           preferred_element_type=jnp.float32)
    bc1_out[...] = _dotT(gin1b[...], ww[...]) + wb[...]


def _stage_a(x, fc0w, fc0b, g0, b0, rm0, rv0,
             gin0w, gin0b, gin1w, gin1b, ww, wb):
    f32 = jnp.float32
    return pl.pallas_call(
        _stage_a_body,
        out_shape=(
            jax.ShapeDtypeStruct((N, HH), f32),
            jax.ShapeDtypeStruct((N, HH), f32),
            jax.ShapeDtypeStruct((H, H), f32),
            jax.ShapeDtypeStruct((1, H), f32),
            jax.ShapeDtypeStruct((H, H), f32),
            jax.ShapeDtypeStruct((1, H), f32),
        ),
    )(x, fc0w, fc0b, g0, b0, rm0, rv0, gin0w, gin0b, gin1w, gin1b, ww, wb)


def _gin_stage_body(hl, hr, aggl, aggr, wc, bc, eps, g, b, rm, rv, *outs):
    e = eps[0, 0]
    yl = (1.0 + e) * hl[...] + aggl[...]
    yr = (1.0 + e) * hr[...] + aggr[...]
    w = wc[...]
    z = _dotT(yl, w[:, :HH]) + _dotT(yr, w[:, HH:]) + bc[...]
    s, t = _bn_scale_shift(g[...], b[...], rm[...], rv[...])
    h = jnp.maximum(z * s + t, 0.0)
    if len(outs) == 2:
        outs[0][...] = h[:, :HH]
        outs[1][...] = h[:, HH:]
    else:
        outs[0][...] = h


def _gin_stage(hl, hr, aggl, aggr, wc, bc, eps, g, b, rm, rv, last):
    f32 = jnp.float32
    if last:
        out_shape = jax.ShapeDtypeStruct((N, H), f32)
    else:
        out_shape = (jax.ShapeDtypeStruct((N, HH), f32),
                     jax.ShapeDtypeStruct((N, HH), f32))
    return pl.pallas_call(
        _gin_stage_body,
        out_shape=out_shape,
    )(hl, hr, aggl, aggr, wc, bc, eps, g, b, rm, rv)


def _segsum_body(hl_hbm, hr_hbm, src_hbm, dst_hbm, outl_hbm, outr_hbm,
                 idxs_v, idxd_v, rows_v, zrow_v, semi, semg, acc_sh):
    c = lax.axis_index("c")
    s = lax.axis_index("s")

    def idx_start(j, m):
        pltpu.async_copy(src_hbm.at[s, j], idxs_v.at[m], semi[m])
        pltpu.async_copy(dst_hbm.at[s, j], idxd_v.at[m], semi[m])

    def idx_wait(m):
        pltpu.make_async_copy(src_hbm.at[s, 0], idxs_v.at[m],
                              semi[m]).wait()
        pltpu.make_async_copy(dst_hbm.at[s, 0], idxd_v.at[m],
                              semi[m]).wait()

    def prologue(h_hbm):
        idx_start(0, 0)
        idx_start(1, 1)
        idx_wait(0)
        pltpu.async_copy(h_hbm.at[idxs_v.at[0]], rows_v.at[0], semg[0])

    @pl.when(c == 0)
    def _():
        prologue(hl_hbm)

    @pl.when(c == 1)
    def _():
        prologue(hr_hbm)

    zv = jnp.zeros((16,), jnp.float32)
    for r in range(32):
        for j in range(HH // 16):
            zrow_v[r, pl.ds(j * 16, 16)] = zv
    base = s * 640
    for i in range(20):
        pltpu.sync_copy(zrow_v, acc_sh.at[pl.ds(base + i * 32, 32)])

    plsc.subcore_barrier()

    def run_edges(h_hbm):
        def outer(jj, carry):
            j0 = 2 * jj
            for b in range(2):
                j = j0 + b
                nb = 1 - b
                idx_wait(nb)
                pltpu.async_copy(h_hbm.at[idxs_v.at[nb]], rows_v.at[nb],
                                 semg[nb])
                pltpu.make_async_copy(h_hbm.at[idxs_v.at[b]], rows_v.at[b],
                                      semg[b]).wait()
                pltpu.sync_copy(rows_v.at[b], acc_sh.at[idxd_v.at[b]],
                                add=True)
                idx_start(j + 2, b)
            return carry

        lax.fori_loop(0, NCHUNK // 2, outer, 0)
        pltpu.make_async_copy(h_hbm.at[idxs_v.at[0]], rows_v.at[0],
                              semg[0]).wait()
        idx_wait(1)

    @pl.when(c == 0)
    def _():
        run_edges(hl_hbm)

    @pl.when(c == 1)
    def _():
        run_edges(hr_hbm)

    plsc.subcore_barrier()

    def copy_out(out_hbm):
        @pl.when(s < 15)
        def _():
            st = s * 640
            pltpu.sync_copy(acc_sh.at[pl.ds(st, 640)],
                            out_hbm.at[pl.ds(st, 640)])

        @pl.when(s == 15)
        def _():
            pltpu.sync_copy(acc_sh.at[pl.ds(9600, 400)],
                            out_hbm.at[pl.ds(9600, 400)])

    @pl.when(c == 0)
    def _():
        copy_out(outl_hbm)

    @pl.when(c == 1)
    def _():
        copy_out(outr_hbm)


@functools.partial(
    pl.kernel,
    out_type=(jax.ShapeDtypeStruct((N, HH), jnp.float32),
              jax.ShapeDtypeStruct((N, HH), jnp.float32)),
    mesh=plsc.VectorSubcoreMesh(core_axis_name="c", subcore_axis_name="s"),
    scratch_types=(
        [
            pltpu.VMEM((2, K), jnp.int32),
            pltpu.VMEM((2, K), jnp.int32),
            pltpu.VMEM((2, K, HH), jnp.float32),
            pltpu.VMEM((32, HH), jnp.float32),
        ]
        + [pltpu.SemaphoreType.DMA] * 4
        + [pltpu.VMEM_SHARED((SP_ROWS, HH), jnp.float32)]
    ),
)
def _segsum(hl, hr, src3, dst3, outl, outr,
            idxs_v, idxd_v, rows_v, zrow_v, *rest):
    semi = rest[:2]
    semg = rest[2:4]
    acc_sh = rest[-1]
    _segsum_body(hl, hr, src3, dst3, outl, outr,
                 idxs_v, idxd_v, rows_v, zrow_v, semi, semg, acc_sh)


def kernel(x, edge_index, fc0_W, fc0_b, gin0_W, gin0_b, gin1_W, gin1_b,
           W_W, W_b, eps0, eps1, bn0_g, bn0_b, bn0_rm, bn0_rv,
           bn1_g, bn1_b, bn1_rm, bn1_rv, bn2_g, bn2_b, bn2_rm, bn2_rv):
    f32 = jnp.float32
    row = lambda v: v.reshape(1, H).astype(f32)

    E = edge_index.shape[1]
    pad = NSUB * EPT - E
    src = jnp.concatenate([edge_index[0], jnp.zeros((pad,), jnp.int32)])
    dst = jnp.concatenate([edge_index[1], jnp.full((pad,), N, jnp.int32)])
    src3 = jnp.concatenate(
        [src.reshape(NSUB, NCHUNK, K),
         jnp.zeros((NSUB, NCHUNK_P - NCHUNK, K), jnp.int32)], axis=1)
    dst3 = jnp.concatenate(
        [dst.reshape(NSUB, NCHUNK, K),
         jnp.full((NSUB, NCHUNK_P - NCHUNK, K), N, jnp.int32)], axis=1)

    h0l, h0r, wc0, bc0, wc1, bc1 = _stage_a(
        x, fc0_W, row(fc0_b), row(bn0_g), row(bn0_b), row(bn0_rm),
        row(bn0_rv), gin0_W, row(gin0_b), gin1_W, row(gin1_b),
        W_W, row(W_b))

    agg0l, agg0r = _segsum(h0l, h0r, src3, dst3)
    h1l, h1r = _gin_stage(h0l, h0r, agg0l, agg0r, wc0, bc0,
                          eps0.reshape(1, 1), row(bn1_g), row(bn1_b),
                          row(bn1_rm), row(bn1_rv), last=False)

    agg1l, agg1r = _segsum(h1l, h1r, src3, dst3)
    h2 = _gin_stage(h1l, h1r, agg1l, agg1r, wc1, bc1,
                    eps1.reshape(1, 1), row(bn2_g), row(bn2_b),
                    row(bn2_rm), row(bn2_rv), last=True)
    return h2
